# baseline (speedup 1.0000x reference)
import functools

import jax
import jax.numpy as jnp
from jax import lax
from jax.experimental import pallas as pl
from jax.experimental.pallas import tpu as pltpu
from jax.experimental.pallas import tpu_sc as plsc

BATCH = 4096
SEQ_LEN = 200
D_MODEL = 64
LANES = 16

NUM_CORES = 2
NUM_SUBCORES = 16
NUM_WORKERS = NUM_CORES * NUM_SUBCORES
TOTAL_ROWS = BATCH * SEQ_LEN
ROWS_PER_WORKER = TOTAL_ROWS // NUM_WORKERS
CHUNK = SEQ_LEN
CHUNKS_PER_WORKER = ROWS_PER_WORKER // CHUNK
NBUF = 4
LEAD = 2

_mesh = plsc.VectorSubcoreMesh(core_axis_name="c", subcore_axis_name="s")


@functools.partial(
    pl.kernel,
    mesh=_mesh,
    compiler_params=pltpu.CompilerParams(use_tc_tiling_on_sc=False),
    out_type=jax.ShapeDtypeStruct((TOTAL_ROWS, D_MODEL), jnp.float32),
    scratch_types=[
        pltpu.VMEM((ROWS_PER_WORKER,), jnp.int32),
        pltpu.VMEM((SEQ_LEN, D_MODEL), jnp.float32),
        pltpu.VMEM((NBUF, CHUNK, D_MODEL), jnp.float32),
        pltpu.SemaphoreType.DMA((NBUF,)),
        pltpu.SemaphoreType.DMA((NBUF,)),
    ],
)
def _emb_kernel(x_hbm, tok_hbm, pos_hbm, out_hbm, idx_v, pos_v, bufs, gsem, ssem):
    cid = lax.axis_index("c")
    sid = lax.axis_index("s")
    wid = sid * NUM_CORES + cid
    base = pl.multiple_of(wid * ROWS_PER_WORKER, 8)

    pltpu.sync_copy(x_hbm.at[pl.ds(base, ROWS_PER_WORKER)], idx_v)
    pltpu.sync_copy(pos_hbm, pos_v)

    def idx_slice(c):
        return idx_v.at[pl.ds(pl.multiple_of(c * CHUNK, 8), CHUNK)]

    def start_gather(c, b):
        pltpu.async_copy(tok_hbm.at[idx_slice(c)], bufs.at[b], gsem.at[b])

    def wait_gather(b):
        pltpu.make_async_copy(
            tok_hbm.at[idx_slice(0)], bufs.at[b], gsem.at[b]
        ).wait()

    def out_slice(c):
        return out_hbm.at[pl.ds(pl.multiple_of(base + c * CHUNK, 8), CHUNK)]

    def start_store(c, b):
        pltpu.async_copy(bufs.at[b], out_slice(c), ssem.at[b])

    def wait_store(b):
        pltpu.make_async_copy(bufs.at[b], out_slice(0), ssem.at[b]).wait()

    def add_pos(b):
        buf = bufs.at[b]

        def row(i, carry):
            for k in range(D_MODEL // LANES):
                sl = pl.ds(k * LANES, LANES)
                buf[i, sl] = buf[i, sl] + pos_v[i, sl]
            return carry

        lax.fori_loop(0, SEQ_LEN, row, 0, unroll=2)

    for b in range(LEAD):
        start_gather(b, b)

    def outer(go, carry):
        for b in range(NBUF):
            c = go * NBUF + b
            nslot = (b + LEAD) % NBUF

            @pl.when(c < CHUNKS_PER_WORKER - LEAD)
            def _issue():
                @pl.when(c >= NBUF - LEAD)
                def _drain():
                    wait_store(nslot)

                start_gather(c + LEAD, nslot)

            wait_gather(b)
            add_pos(b)
            start_store(c, b)
        return carry

    lax.fori_loop(0, CHUNKS_PER_WORKER // NBUF, outer, 0)

    for b in range(NBUF):
        wait_store(b)


@jax.jit
def kernel(x, token_table, pos_table):
    x_flat = x.reshape(-1).astype(jnp.int32)
    out = _emb_kernel(x_flat, token_table, pos_table)
    return out.reshape(BATCH, SEQ_LEN, D_MODEL)
